# Initial kernel scaffold; baseline (speedup 1.0000x reference)
import jax
import jax.numpy as jnp
from jax.experimental import pallas as pl


def kernel(x, token_table, pos_table):
    raise NotImplementedError("write your pallas kernel here")


# speedup vs baseline: 1.5121x; 1.5121x over previous
import functools

import jax
import jax.numpy as jnp
from jax import lax
from jax.experimental import pallas as pl
from jax.experimental.pallas import tpu as pltpu
from jax.experimental.pallas import tpu_sc as plsc

_NUM_CORES = 2
_NUM_SUBCORES = 16
_LANES = 16


def kernel(x, token_table, pos_table):
    B, S = x.shape
    V, D = token_table.shape
    C = S // 2
    n_chunks = (B * S) // C
    nw = _NUM_CORES * _NUM_SUBCORES
    chunks_per_w = n_chunks // nw

    idx = x.reshape(n_chunks, C).astype(jnp.int32)

    mesh = plsc.VectorSubcoreMesh(core_axis_name="c", subcore_axis_name="s")

    @functools.partial(
        pl.kernel,
        mesh=mesh,
        out_type=jax.ShapeDtypeStruct((n_chunks, C, D), jnp.float32),
        scratch_types=[
            pltpu.VMEM((chunks_per_w, C), jnp.int32),
            pltpu.VMEM((S, D), jnp.float32),
            pltpu.VMEM((C, D), jnp.float32),
            pltpu.SemaphoreType.DMA,
        ],
    )
    def emb_kernel(idx_hbm, tok_hbm, pos_hbm, out_hbm, idx_v, pos_v, buf, sem):
        wid = lax.axis_index("s") * _NUM_CORES + lax.axis_index("c")
        base = wid * chunks_per_w
        pltpu.sync_copy(pos_hbm, pos_v)
        pltpu.sync_copy(idx_hbm.at[pl.ds(base, chunks_per_w)], idx_v)

        def chunk_body(k, carry):
            pltpu.async_copy(tok_hbm.at[idx_v.at[k]], buf, sem).wait()
            phase = (k % 2) * C

            def row_body(i, carry2):
                for j in range(D // _LANES):
                    sl = pl.ds(j * _LANES, _LANES)
                    buf[i, sl] = buf[i, sl] + pos_v[phase + i, sl]
                return carry2

            lax.fori_loop(0, C, row_body, 0)
            pltpu.sync_copy(buf, out_hbm.at[base + k])
            return carry

        lax.fori_loop(0, chunks_per_w, chunk_body, 0)

    out = emb_kernel(idx, token_table, pos_table)
    return out.reshape(B, S, D)


# speedup vs baseline: 3.9555x; 2.6159x over previous
import functools

import jax
import jax.numpy as jnp
from jax import lax
from jax.experimental import pallas as pl
from jax.experimental.pallas import tpu as pltpu
from jax.experimental.pallas import tpu_sc as plsc

_NUM_CORES = 2
_NUM_SUBCORES = 16
_LANES = 16
_NBUF = 4


def kernel(x, token_table, pos_table):
    B, S = x.shape
    V, D = token_table.shape
    C = S // 2
    n_chunks = (B * S) // C
    nw = _NUM_CORES * _NUM_SUBCORES
    chunks_per_w = n_chunks // nw
    n_steps = chunks_per_w // _NBUF

    idx = x.reshape(n_chunks, C).astype(jnp.int32)

    mesh = plsc.VectorSubcoreMesh(core_axis_name="c", subcore_axis_name="s")

    @functools.partial(
        pl.kernel,
        mesh=mesh,
        out_type=jax.ShapeDtypeStruct((n_chunks, C, D), jnp.float32),
        scratch_types=[
            pltpu.VMEM((chunks_per_w, C), jnp.int32),
            pltpu.VMEM((S, D), jnp.float32),
            [pltpu.VMEM((C, D), jnp.float32)] * _NBUF,
            [pltpu.SemaphoreType.DMA] * _NBUF,
            [pltpu.SemaphoreType.DMA] * _NBUF,
        ],
    )
    def emb_kernel(idx_hbm, tok_hbm, pos_hbm, out_hbm, idx_v, pos_v, bufs,
                   gsems, ssems):
        wid = lax.axis_index("s") * _NUM_CORES + lax.axis_index("c")
        base = wid * chunks_per_w
        pltpu.sync_copy(pos_hbm, pos_v)
        pltpu.sync_copy(idx_hbm.at[pl.ds(base, chunks_per_w)], idx_v)

        def gather(kk, b):
            return pltpu.make_async_copy(
                tok_hbm.at[idx_v.at[kk]], bufs[b], gsems[b])

        def store(kk, b):
            return pltpu.make_async_copy(
                bufs[b], out_hbm.at[base + kk], ssems[b])

        gather(0, 0).start()
        gather(1, 1).start()

        def step_body(k, carry):
            for b in range(_NBUF):
                kk = k * _NBUF + b
                gather(kk, b).wait()

                phase = (b % 2) * C
                buf = bufs[b]

                @plsc.parallel_loop(0, C, step=1, unroll=2)
                def row_add(i):
                    vals = [
                        buf[i, pl.ds(j * _LANES, _LANES)]
                        + pos_v[phase + i, pl.ds(j * _LANES, _LANES)]
                        for j in range(D // _LANES)
                    ]
                    for j in range(D // _LANES):
                        buf[i, pl.ds(j * _LANES, _LANES)] = vals[j]

                store(kk, b).start()

                b2 = (b + 2) % _NBUF

                @pl.when(kk >= 2)
                def _wait_prev_store():
                    store(kk - 2, b2).wait()

                @pl.when(kk + 2 < chunks_per_w)
                def _issue_next_gather():
                    gather(kk + 2, b2).start()
            return carry

        lax.fori_loop(0, n_steps, step_body, 0)

        store(chunks_per_w - 2, (chunks_per_w - 2) % _NBUF).wait()
        store(chunks_per_w - 1, (chunks_per_w - 1) % _NBUF).wait()

    out = emb_kernel(idx, token_table, pos_table)
    return out.reshape(B, S, D)


# speedup vs baseline: 8.4423x; 2.1343x over previous
import functools

import jax
import jax.numpy as jnp
from jax import lax
from jax.experimental import pallas as pl
from jax.experimental.pallas import tpu as pltpu
from jax.experimental.pallas import tpu_sc as plsc

_NUM_CORES = 2
_NUM_SUBCORES = 16
_LANES = 16
_NBUF = 4
_C = 128


def kernel(x, token_table, pos_table):
    B, S = x.shape
    V, D = token_table.shape
    n_tok = B * S
    n_chunks = n_tok // _C
    nw = _NUM_CORES * _NUM_SUBCORES
    chunks_per_w = n_chunks // nw
    n_steps = chunks_per_w // _NBUF

    idx = x.reshape(n_chunks, _C).astype(jnp.int32)

    mesh = plsc.VectorSubcoreMesh(core_axis_name="c", subcore_axis_name="s")

    @functools.partial(
        pl.kernel,
        mesh=mesh,
        out_type=jax.ShapeDtypeStruct((n_tok, D), jnp.float32),
        scratch_types=[
            pltpu.VMEM((chunks_per_w, _C), jnp.int32),
            pltpu.VMEM((S, D), jnp.float32),
            [pltpu.VMEM((_C, D), jnp.float32)] * _NBUF,
            [pltpu.SemaphoreType.DMA] * _NBUF,
            [pltpu.SemaphoreType.DMA] * _NBUF,
        ],
    )
    def emb_kernel(idx_hbm, tok_hbm, pos_hbm, out_hbm, idx_v, pos_v, bufs,
                   gsems, ssems):
        wid = lax.axis_index("s") * _NUM_CORES + lax.axis_index("c")
        base = wid * chunks_per_w
        pltpu.sync_copy(pos_hbm, pos_v)
        pltpu.sync_copy(idx_hbm.at[pl.ds(base, chunks_per_w)], idx_v)

        def gather(kk, b):
            return pltpu.make_async_copy(
                tok_hbm.at[idx_v.at[kk]], bufs[b], gsems[b])

        def store(kk, b):
            return pltpu.make_async_copy(
                bufs[b], out_hbm.at[pl.ds((base + kk) * _C, _C)], ssems[b])

        gather(0, 0).start()
        gather(1, 1).start()

        def step_body(k, carry):
            for b in range(_NBUF):
                kk = k * _NBUF + b
                gather(kk, b).wait()

                start = ((base + kk) * _C) % S
                buf = bufs[b]

                @plsc.parallel_loop(0, _C, step=1, unroll=2)
                def row_add(i):
                    r = start + i
                    r = r - jnp.where(r >= S, S, 0)
                    vals = [
                        buf[i, pl.ds(j * _LANES, _LANES)]
                        + pos_v[r, pl.ds(j * _LANES, _LANES)]
                        for j in range(D // _LANES)
                    ]
                    for j in range(D // _LANES):
                        buf[i, pl.ds(j * _LANES, _LANES)] = vals[j]

                store(kk, b).start()

                b2 = (b + 2) % _NBUF

                @pl.when(kk >= 2)
                def _wait_prev_store():
                    store(kk - 2, b2).wait()

                @pl.when(kk + 2 < chunks_per_w)
                def _issue_next_gather():
                    gather(kk + 2, b2).start()
            return carry

        lax.fori_loop(0, n_steps, step_body, 0)

        store(chunks_per_w - 2, (chunks_per_w - 2) % _NBUF).wait()
        store(chunks_per_w - 1, (chunks_per_w - 1) % _NBUF).wait()

    out = emb_kernel(idx, token_table, pos_table)
    return out.reshape(B, S, D)


# speedup vs baseline: 8.9166x; 1.0562x over previous
import functools

import jax
import jax.numpy as jnp
from jax import lax
from jax.experimental import pallas as pl
from jax.experimental.pallas import tpu as pltpu
from jax.experimental.pallas import tpu_sc as plsc

_NUM_CORES = 2
_NUM_SUBCORES = 16
_LANES = 16
_NBUF = 4
_C = 128


def kernel(x, token_table, pos_table):
    B, S = x.shape
    V, D = token_table.shape
    n_tok = B * S
    n_chunks = n_tok // _C
    nw = _NUM_CORES * _NUM_SUBCORES
    chunks_per_w = n_chunks // nw
    n_steps = chunks_per_w // _NBUF

    idx = x.reshape(n_chunks, _C).astype(jnp.int32)

    mesh = plsc.VectorSubcoreMesh(core_axis_name="c", subcore_axis_name="s")

    @functools.partial(
        pl.kernel,
        mesh=mesh,
        out_type=jax.ShapeDtypeStruct((n_tok, D), jnp.float32),
        scratch_types=[
            pltpu.VMEM((chunks_per_w, _C), jnp.int32),
            pltpu.VMEM((S, D), jnp.float32),
            [pltpu.VMEM((_C, D), jnp.float32)] * _NBUF,
            [pltpu.SemaphoreType.DMA] * (2 * _NBUF),
            [pltpu.SemaphoreType.DMA] * _NBUF,
        ],
    )
    def emb_kernel(idx_hbm, tok_hbm, pos_hbm, out_hbm, idx_v, pos_v, bufs,
                   gsems, ssems):
        wid = lax.axis_index("s") * _NUM_CORES + lax.axis_index("c")
        base = wid * chunks_per_w
        pltpu.sync_copy(pos_hbm, pos_v)
        pltpu.sync_copy(idx_hbm.at[pl.ds(base, chunks_per_w)], idx_v)

        H = _C // 2

        def gather_half(kk, b, h):
            return pltpu.make_async_copy(
                tok_hbm.at[idx_v.at[kk, pl.ds(h * H, H)]],
                bufs[b].at[pl.ds(h * H, H)],
                gsems[2 * b + h])

        def gather_start(kk, b):
            gather_half(kk, b, 0).start()
            gather_half(kk, b, 1).start()

        def gather_wait(kk, b):
            gather_half(kk, b, 0).wait()
            gather_half(kk, b, 1).wait()

        def store(kk, b):
            return pltpu.make_async_copy(
                bufs[b], out_hbm.at[pl.ds((base + kk) * _C, _C)], ssems[b])

        gather_start(0, 0)
        gather_start(1, 1)

        def step_body(k, carry):
            for b in range(_NBUF):
                kk = k * _NBUF + b
                gather_wait(kk, b)

                b2 = (b + 2) % _NBUF

                @pl.when(kk >= 2)
                def _wait_prev_store():
                    store(kk - 2, b2).wait()

                @pl.when(kk + 2 < chunks_per_w)
                def _issue_next_gather():
                    gather_start(kk + 2, b2)

                start = ((base + kk) * _C) % S
                buf = bufs[b]

                @plsc.parallel_loop(0, _C, step=1, unroll=4)
                def row_add(i):
                    r = start + i
                    r = r - jnp.where(r >= S, S, 0)
                    vals = [
                        buf[i, pl.ds(j * _LANES, _LANES)]
                        + pos_v[r, pl.ds(j * _LANES, _LANES)]
                        for j in range(D // _LANES)
                    ]
                    for j in range(D // _LANES):
                        buf[i, pl.ds(j * _LANES, _LANES)] = vals[j]

                store(kk, b).start()
            return carry

        lax.fori_loop(0, n_steps, step_body, 0)

        store(chunks_per_w - 2, (chunks_per_w - 2) % _NBUF).wait()
        store(chunks_per_w - 1, (chunks_per_w - 1) % _NBUF).wait()

    out = emb_kernel(idx, token_table, pos_table)
    return out.reshape(B, S, D)
